# Initial kernel scaffold; baseline (speedup 1.0000x reference)
import jax
import jax.numpy as jnp
from jax.experimental import pallas as pl


def kernel(h, x, edge_index, We, be, ew1, eb1, ew2, eb2, nw1, nb1, nw2, nb2, cw1, cb1, cw2, cb2):
    raise NotImplementedError("write your pallas kernel here")


# speedup vs baseline: 2.0840x; 2.0840x over previous
import functools

import jax
import jax.numpy as jnp
from jax import lax
from jax.experimental import pallas as pl
from jax.experimental.pallas import tpu as pltpu
from jax.experimental.pallas import tpu_sc as plsc

N = 10000
E = 320000
H = 128
NP = 10240
EP = 327680
NC = 2
NS = 16
NWK = NC * NS
EPW = EP // NWK
C = 128
CH = EPW // C
BAND = NP // NS
F32 = jnp.float32
BF16 = jnp.bfloat16
I32 = jnp.int32
U32 = jnp.uint32

_mesh = plsc.VectorSubcoreMesh(core_axis_name="c", subcore_axis_name="s")


def _bf16_round(v):
    u = plsc.bitcast(v, U32)
    lsb = (u >> 16) & jnp.uint32(1)
    u2 = (u + jnp.uint32(0x7FFF) + lsb) & jnp.uint32(0xFFFF0000)
    return plsc.bitcast(u2, F32)


@functools.partial(
    pl.kernel,
    mesh=_mesh,
    compiler_params=pltpu.CompilerParams(needs_layout_passes=False),
    out_type=[
        jax.ShapeDtypeStruct((EP, 128), F32),
        jax.ShapeDtypeStruct((EP,), F32),
        jax.ShapeDtypeStruct((EP,), F32),
        jax.ShapeDtypeStruct((EP,), F32),
    ],
    scratch_types=[
        pltpu.VMEM((C, 128), F32),
        pltpu.VMEM((C, 128), F32),
        pltpu.VMEM((NP * 3,), F32),
        pltpu.VMEM((C,), F32),
        pltpu.VMEM((C,), F32),
        pltpu.VMEM((C,), F32),
        pltpu.VMEM((C,), F32),
        pltpu.VMEM((128,), F32),
        pltpu.VMEM((C,), I32),
        pltpu.VMEM((C,), I32),
        pltpu.SemaphoreType.DMA,
        pltpu.SemaphoreType.DMA,
    ],
)
def _sc1(a_hbm, b_hbm, xt3_hbm, row_hbm, col_hbm, w1d_hbm,
         r1_hbm, r2x_hbm, r2y_hbm, r2z_hbm,
         ga, gb, xt_tile, pxbuf, pybuf, pzbuf, dbuf, w1dv,
         idxr, idxc, sa, sb):
    cid = lax.axis_index("c")
    sid = lax.axis_index("s")
    wid = sid * NC + cid

    pltpu.sync_copy(w1d_hbm, w1dv)
    pltpu.sync_copy(xt3_hbm, xt_tile)
    for j in range(8):
        fs = pl.ds(j * 16, 16)
        w1dv[fs] = _bf16_round(w1dv[fs])

    ebase = wid * EPW

    def chunk(k, carry):
        base = ebase + k * C
        sl = pl.ds(base, C)
        pltpu.sync_copy(row_hbm.at[sl], idxr)
        pltpu.sync_copy(col_hbm.at[sl], idxc)
        cpa = pltpu.async_copy(a_hbm.at[idxr], ga, sa)
        cpb = pltpu.async_copy(b_hbm.at[idxc], gb, sb)

        for g in range(C // 16):
            gsl = pl.ds(g * 16, 16)
            rv = idxr[gsl] * 3
            cv = idxc[gsl] * 3
            px = plsc.load_gather(xt_tile, [rv]) - plsc.load_gather(xt_tile, [cv])
            py = plsc.load_gather(xt_tile, [rv + 1]) - plsc.load_gather(xt_tile, [cv + 1])
            pz = plsc.load_gather(xt_tile, [rv + 2]) - plsc.load_gather(xt_tile, [cv + 2])
            pxbuf[gsl] = px
            pybuf[gsl] = py
            pzbuf[gsl] = pz
            dbuf[gsl] = _bf16_round(px * px + py * py + pz * pz)

        cpa.wait()
        cpb.wait()

        def edge(e, c2):
            dv = plsc.load_gather(dbuf, [jnp.full((16,), e, I32)])
            for j in range(8):
                fs = pl.ds(j * 16, 16)
                z = ga[e, fs] + gb[e, fs] + dv * w1dv[fs]
                ga[e, fs] = jnp.maximum(z, 0.0)
            return c2

        lax.fori_loop(0, C, edge, 0)

        pltpu.sync_copy(ga, r1_hbm.at[sl])
        pltpu.sync_copy(pxbuf, r2x_hbm.at[sl])
        pltpu.sync_copy(pybuf, r2y_hbm.at[sl])
        pltpu.sync_copy(pzbuf, r2z_hbm.at[sl])
        return carry

    lax.fori_loop(0, CH, chunk, 0)


@functools.partial(
    pl.kernel,
    mesh=_mesh,
    compiler_params=pltpu.CompilerParams(needs_layout_passes=False),
    out_type=[jax.ShapeDtypeStruct((NC, NP, 128), F32)],
    scratch_types=[
        pltpu.VMEM((C, 128), F32),
        pltpu.VMEM((C,), I32),
        pltpu.VMEM_SHARED((NP, 128), F32),
    ],
)
def _sc2a(e_hbm, row_hbm, z128_hbm, s_hbm, ebuf, idxr, ssh):
    cid = lax.axis_index("c")
    sid = lax.axis_index("s")
    wid = sid * NC + cid

    band = pl.ds(sid * BAND, BAND)
    pltpu.sync_copy(z128_hbm.at[band], ssh.at[band])
    plsc.subcore_barrier()

    ebase = wid * EPW

    def chunk(k, carry):
        base = ebase + k * C
        sl = pl.ds(base, C)
        pltpu.sync_copy(e_hbm.at[sl], ebuf)
        pltpu.sync_copy(row_hbm.at[sl], idxr)
        pltpu.sync_copy(ebuf, ssh.at[idxr], add=True)
        return carry

    lax.fori_loop(0, CH, chunk, 0)

    plsc.subcore_barrier()
    pltpu.sync_copy(ssh.at[band], s_hbm.at[cid, band])


@functools.partial(
    pl.kernel,
    mesh=_mesh,
    compiler_params=pltpu.CompilerParams(needs_layout_passes=False),
    out_type=[jax.ShapeDtypeStruct((NC, NP, 128), F32)],
    scratch_types=[
        pltpu.VMEM((C,), F32),
        pltpu.VMEM((C,), F32),
        pltpu.VMEM((C,), F32),
        pltpu.VMEM((C,), F32),
        pltpu.VMEM((C, 128), F32),
        pltpu.VMEM((48,), F32),
        pltpu.VMEM((C,), I32),
        pltpu.VMEM_SHARED((NP, 128), F32),
    ],
)
def _sc2b(alpha_hbm, r2x_hbm, r2y_hbm, r2z_hbm, row_hbm, z128_hbm, mask_hbm,
          xacc_hbm, abuf, pxb, pyb, pzb, obuf, mv, idxr, accsh):
    cid = lax.axis_index("c")
    sid = lax.axis_index("s")
    wid = sid * NC + cid

    band = pl.ds(sid * BAND, BAND)
    pltpu.sync_copy(z128_hbm.at[band], accsh.at[band])
    pltpu.sync_copy(z128_hbm.at[pl.ds(0, C)], obuf)
    pltpu.sync_copy(mask_hbm, mv)
    plsc.subcore_barrier()

    m1 = mv[0:16]
    m2 = mv[16:32]
    m3 = mv[32:48]

    ebase = wid * EPW

    def chunk(k, carry):
        base = ebase + k * C
        sl = pl.ds(base, C)
        pltpu.sync_copy(alpha_hbm.at[sl], abuf)
        pltpu.sync_copy(r2x_hbm.at[sl], pxb)
        pltpu.sync_copy(r2y_hbm.at[sl], pyb)
        pltpu.sync_copy(r2z_hbm.at[sl], pzb)
        pltpu.sync_copy(row_hbm.at[sl], idxr)

        def edge(e, c2):
            ev = jnp.full((16,), e, I32)
            bx = plsc.load_gather(pxb, [ev])
            by = plsc.load_gather(pyb, [ev])
            bz = plsc.load_gather(pzb, [ev])
            ba = plsc.load_gather(abuf, [ev])
            obuf[e, 0:16] = ba * (bx * m1 + by * m2 + bz * m3)
            return c2

        lax.fori_loop(0, C, edge, 0)

        pltpu.sync_copy(obuf, accsh.at[idxr], add=True)
        return carry

    lax.fori_loop(0, CH, chunk, 0)

    plsc.subcore_barrier()
    pltpu.sync_copy(accsh.at[band], xacc_hbm.at[cid, band])


BM = 1280
BE = 8192


def _bdot(a, b):
    return jnp.dot(a.astype(BF16), b.astype(BF16),
                   preferred_element_type=F32)


def _tck1_body(h_ref, we_ref, be_ref, ew1a_ref, ew1b_ref, eb1_ref,
               h1_ref, a_ref, b_ref):
    h1 = _bdot(h_ref[...], we_ref[...]) + be_ref[...]
    h1_ref[...] = h1
    heb1 = 0.5 * eb1_ref[...]
    a_ref[...] = _bdot(h1, ew1a_ref[...]) + heb1
    b_ref[...] = _bdot(h1, ew1b_ref[...]) + heb1


def _tck2_body(r_ref, ew2_ref, eb2_ref, cw1_ref, cb1_ref, cw2_ref, cb2_ref,
               e_ref, out_ref):
    e_ij = _bdot(r_ref[...], ew2_ref[...]) + eb2_ref[...]
    e_ref[...] = e_ij
    t = jnp.maximum(_bdot(e_ij, cw1_ref[...]) + cb1_ref[...], 0.0)
    a = _bdot(t, cw2_ref[...]) + cb2_ref[...]
    out_ref[...] = a.reshape(BE // 128, 128)


def _tck3_body(last, h_ref, xt_ref, s128_ref, xacc_ref,
               nw1a_ref, nw1b_ref, nb1_ref, nw2_ref,
               nb2_ref, ew1a_ref, ew1b_ref, eb1_ref, *out_refs):
    h = h_ref[...]
    m = s128_ref[0] + s128_ref[1]
    t = jnp.maximum(_bdot(h, nw1a_ref[...]) + _bdot(m, nw1b_ref[...])
                    + nb1_ref[...], 0.0)
    hn = h + _bdot(t, nw2_ref[...]) + nb2_ref[...]
    out_refs[0][...] = hn
    lane = lax.broadcasted_iota(I32, (1, 128), 1)
    mask = jnp.where((lane >= 1) & (lane <= 3), 1.0, 0.0).astype(F32)
    out_refs[1][...] = (xt_ref[...] + xacc_ref[0] + xacc_ref[1]) * mask
    if not last:
        heb1 = 0.5 * eb1_ref[...]
        out_refs[2][...] = _bdot(hn, ew1a_ref[...]) + heb1
        out_refs[3][...] = _bdot(hn, ew1b_ref[...]) + heb1


def _row_spec(bm, w):
    return pl.BlockSpec((bm, w), lambda i: (i, 0))


def _full_spec(shape):
    return pl.BlockSpec(shape, lambda i: tuple(0 for _ in shape))


def _pair_spec(bm, w):
    return pl.BlockSpec((NC, bm, w), lambda i: (0, i, 0))


def _tck1(h_pad, We, be, ew1a, ew1b, eb1):
    return pl.pallas_call(
        _tck1_body,
        grid=(NP // BM,),
        in_specs=[
            _row_spec(BM, 128),
            _full_spec((128, 128)), _full_spec((1, 128)),
            _full_spec((128, 128)), _full_spec((128, 128)),
            _full_spec((1, 128)),
        ],
        out_specs=[
            _row_spec(BM, 128), _row_spec(BM, 128), _row_spec(BM, 128),
        ],
        out_shape=[
            jax.ShapeDtypeStruct((NP, 128), F32),
            jax.ShapeDtypeStruct((NP, 128), F32),
            jax.ShapeDtypeStruct((NP, 128), F32),
        ],
    )(h_pad, We, be, ew1a, ew1b, eb1)


def _tck2(r1, ew2, eb2, cw1, cb1, cw2, cb2):
    return pl.pallas_call(
        _tck2_body,
        grid=(EP // BE,),
        in_specs=[
            _row_spec(BE, 128),
            _full_spec((128, 128)), _full_spec((1, 128)),
            _full_spec((128, 128)), _full_spec((1, 128)),
            _full_spec((128, 1)), _full_spec((1, 1)),
        ],
        out_specs=[_row_spec(BE, 128), _row_spec(BE // 128, 128)],
        out_shape=[
            jax.ShapeDtypeStruct((EP, 128), F32),
            jax.ShapeDtypeStruct((EP // 128, 128), F32),
        ],
    )(r1, ew2, eb2, cw1, cb1, cw2, cb2)


def _tck3(last, h1, xt, s128o, xacc, nw1a, nw1b, nb1, nw2,
          nb2, ew1a, ew1b, eb1):
    out_specs = [_row_spec(BM, 128), _row_spec(BM, 128)]
    out_shape = [
        jax.ShapeDtypeStruct((NP, 128), F32),
        jax.ShapeDtypeStruct((NP, 128), F32),
    ]
    if not last:
        out_specs += [_row_spec(BM, 128), _row_spec(BM, 128)]
        out_shape += [
            jax.ShapeDtypeStruct((NP, 128), F32),
            jax.ShapeDtypeStruct((NP, 128), F32),
        ]
    return pl.pallas_call(
        functools.partial(_tck3_body, last),
        grid=(NP // BM,),
        in_specs=[
            _row_spec(BM, 128), _row_spec(BM, 128),
            _pair_spec(BM, 128), _pair_spec(BM, 128),
            _full_spec((128, 128)), _full_spec((128, 128)),
            _full_spec((1, 128)), _full_spec((128, 128)),
            _full_spec((1, 128)),
            _full_spec((128, 128)), _full_spec((128, 128)),
            _full_spec((1, 128)),
        ],
        out_specs=out_specs,
        out_shape=out_shape,
    )(h1, xt, s128o, xacc, nw1a, nw1b, nb1, nw2, nb2,
      ew1a, ew1b, eb1)


def kernel(h, x, edge_index, We, be, ew1, eb1, ew2, eb2,
           nw1, nb1, nw2, nb2, cw1, cb1, cw2, cb2):
    row, col = edge_index[0], edge_index[1]
    rowp = jnp.concatenate([row, jnp.full((EP - E,), N, I32)])
    colp = jnp.concatenate([col, jnp.full((EP - E,), N, I32)])
    h_pad = jnp.pad(h, ((0, NP - N), (0, 0)))
    xt = jnp.pad(x, ((0, NP - N), (1, 124)))

    ew1a, ew1b, w1d = ew1[:H], ew1[H:2 * H], ew1[2 * H]
    nw1a, nw1b = nw1[:H], nw1[H:]
    be2 = be.reshape(1, 128)
    eb1_2 = eb1.reshape(1, 128)
    eb2_2 = eb2.reshape(1, 128)
    nb1_2 = nb1.reshape(1, 128)
    nb2_2 = nb2.reshape(1, 128)
    cb1_2 = cb1.reshape(1, 128)
    cb2_2 = cb2.reshape(1, 1)
    z128 = jnp.zeros((NP, 128), F32)
    lane_masks = jnp.zeros((48,), F32).at[jnp.array([1, 18, 35])].set(1.0)

    h1, A, B = _tck1(h_pad, We, be2, ew1a, ew1b, eb1_2)

    for layer in range(3):
        xt3 = xt[:, 1:4].reshape(NP * 3)
        r1, r2x, r2y, r2z = _sc1(A, B, xt3, rowp, colp, w1d)
        e_ij, alpha = _tck2(r1, ew2, eb2_2, cw1, cb1_2, cw2, cb2_2)
        s128o, = _sc2a(e_ij, rowp, z128)
        xacc, = _sc2b(alpha.reshape(EP), r2x, r2y, r2z, rowp, z128, lane_masks)
        last = layer == 2
        outs = _tck3(last, h1, xt, s128o, xacc,
                     nw1a, nw1b, nb1_2, nw2, nb2_2, ew1a, ew1b, eb1_2)
        if last:
            h1, xt = outs
        else:
            h1, xt, A, B = outs

    return h1[:N], xt[:N, 1:4]


# speedup vs baseline: 3.6758x; 1.7639x over previous
import functools

import jax
import jax.numpy as jnp
from jax import lax
from jax.experimental import pallas as pl
from jax.experimental.pallas import tpu as pltpu
from jax.experimental.pallas import tpu_sc as plsc

N = 10000
E = 320000
H = 128
NP = 10240
EP = 327680
NC = 2
NS = 16
NWK = NC * NS
EPW = EP // NWK
C = 128
CH = EPW // C
BAND = NP // NS
F32 = jnp.float32
BF16 = jnp.bfloat16
I32 = jnp.int32
U32 = jnp.uint32

_mesh = plsc.VectorSubcoreMesh(core_axis_name="c", subcore_axis_name="s")


def _bf16_round(v):
    u = plsc.bitcast(v, U32)
    lsb = (u >> 16) & jnp.uint32(1)
    u2 = (u + jnp.uint32(0x7FFF) + lsb) & jnp.uint32(0xFFFF0000)
    return plsc.bitcast(u2, F32)


@functools.partial(
    pl.kernel,
    mesh=_mesh,
    compiler_params=pltpu.CompilerParams(needs_layout_passes=False),
    out_type=[
        jax.ShapeDtypeStruct((EP, 128), F32),
        jax.ShapeDtypeStruct((EP,), F32),
        jax.ShapeDtypeStruct((EP,), F32),
        jax.ShapeDtypeStruct((EP,), F32),
    ],
    scratch_types=[
        pltpu.VMEM((C, 128), F32),
        pltpu.VMEM((C, 128), F32),
        pltpu.VMEM((C, 128), F32),
        pltpu.VMEM((C, 128), F32),
        pltpu.VMEM((NP * 3,), F32),
        pltpu.VMEM((C,), F32),
        pltpu.VMEM((C,), F32),
        pltpu.VMEM((C,), F32),
        pltpu.VMEM((C,), F32),
        pltpu.VMEM((128,), F32),
        pltpu.VMEM((CH, C), I32),
        pltpu.VMEM((CH, C), I32),
        pltpu.SemaphoreType.DMA,
        pltpu.SemaphoreType.DMA,
        pltpu.SemaphoreType.DMA,
        pltpu.SemaphoreType.DMA,
    ],
)
def _sc1(a_hbm, b_hbm, xt3_hbm, row_hbm, col_hbm, w1d_hbm,
         r1_hbm, r2x_hbm, r2y_hbm, r2z_hbm,
         ga0, gb0, ga1, gb1, xt_tile, pxbuf, pybuf, pzbuf, dbuf, w1dv,
         idx2r, idx2c, sa0, sb0, sa1, sb1):
    cid = lax.axis_index("c")
    sid = lax.axis_index("s")
    wid = sid * NC + cid

    pltpu.sync_copy(w1d_hbm, w1dv)
    pltpu.sync_copy(xt3_hbm, xt_tile)
    wblk = pl.ds(wid * CH, CH)
    pltpu.sync_copy(row_hbm.at[wblk], idx2r)
    pltpu.sync_copy(col_hbm.at[wblk], idx2c)
    for j in range(8):
        fs = pl.ds(j * 16, 16)
        w1dv[fs] = _bf16_round(w1dv[fs])

    ebase = wid * EPW
    bufs = ((ga0, gb0, sa0, sb0), (ga1, gb1, sa1, sb1))

    def start(k, b):
        ga, gb, sa, sb = bufs[b]
        pltpu.async_copy(a_hbm.at[idx2r.at[k]], ga, sa)
        pltpu.async_copy(b_hbm.at[idx2c.at[k]], gb, sb)

    def finish(k, b):
        ga, gb, sa, sb = bufs[b]
        pltpu.make_async_copy(a_hbm.at[idx2r.at[k]], ga, sa).wait()
        pltpu.make_async_copy(b_hbm.at[idx2c.at[k]], gb, sb).wait()

    def process(k, b):
        ga, gb, _, _ = bufs[b]
        for g in range(C // 16):
            gsl = pl.ds(g * 16, 16)
            rv = idx2r[k, gsl] * 3
            cv = idx2c[k, gsl] * 3
            px = plsc.load_gather(xt_tile, [rv]) - plsc.load_gather(xt_tile, [cv])
            py = plsc.load_gather(xt_tile, [rv + 1]) - plsc.load_gather(xt_tile, [cv + 1])
            pz = plsc.load_gather(xt_tile, [rv + 2]) - plsc.load_gather(xt_tile, [cv + 2])
            pxbuf[gsl] = px
            pybuf[gsl] = py
            pzbuf[gsl] = pz
            dbuf[gsl] = _bf16_round(px * px + py * py + pz * pz)

        finish(k, b)

        def edge(e, c2):
            dv = plsc.load_gather(dbuf, [jnp.full((16,), e, I32)])
            for j in range(8):
                fs = pl.ds(j * 16, 16)
                z = ga[e, fs] + gb[e, fs] + dv * w1dv[fs]
                ga[e, fs] = jnp.maximum(z, 0.0)
            return c2

        lax.fori_loop(0, C, edge, 0)

        sl = pl.ds(ebase + k * C, C)
        pltpu.sync_copy(ga, r1_hbm.at[sl])
        pltpu.sync_copy(pxbuf, r2x_hbm.at[sl])
        pltpu.sync_copy(pybuf, r2y_hbm.at[sl])
        pltpu.sync_copy(pzbuf, r2z_hbm.at[sl])

    start(0, 0)

    def body(i, carry):
        k0 = 2 * i
        start(k0 + 1, 1)
        process(k0, 0)
        start(jnp.remainder(k0 + 2, CH), 0)
        process(k0 + 1, 1)
        return carry

    lax.fori_loop(0, CH // 2, body, 0)
    finish(0, 0)


@functools.partial(
    pl.kernel,
    mesh=_mesh,
    compiler_params=pltpu.CompilerParams(needs_layout_passes=False),
    out_type=[jax.ShapeDtypeStruct((NC, NP, 128), F32)],
    scratch_types=[
        pltpu.VMEM((C, 128), F32),
        pltpu.VMEM((C, 128), F32),
        pltpu.VMEM((CH, C), I32),
        pltpu.VMEM_SHARED((NP, 128), F32),
        pltpu.SemaphoreType.DMA,
        pltpu.SemaphoreType.DMA,
    ],
)
def _sc2a(e_hbm, row_hbm, z128_hbm, s_hbm, ebuf0, ebuf1, idx2r, ssh, se0, se1):
    cid = lax.axis_index("c")
    sid = lax.axis_index("s")
    wid = sid * NC + cid

    band = pl.ds(sid * BAND, BAND)
    pltpu.sync_copy(z128_hbm.at[band], ssh.at[band])
    pltpu.sync_copy(row_hbm.at[pl.ds(wid * CH, CH)], idx2r)
    plsc.subcore_barrier()

    ebase = wid * EPW
    bufs = ((ebuf0, se0), (ebuf1, se1))

    def start(k, b):
        eb, se = bufs[b]
        pltpu.async_copy(e_hbm.at[pl.ds(ebase + k * C, C)], eb, se)

    def process(k, b):
        eb, se = bufs[b]
        pltpu.make_async_copy(e_hbm.at[pl.ds(ebase + k * C, C)], eb, se).wait()
        pltpu.sync_copy(eb, ssh.at[idx2r.at[k]], add=True)

    start(0, 0)

    def body(i, carry):
        k0 = 2 * i
        start(k0 + 1, 1)
        process(k0, 0)
        start(jnp.remainder(k0 + 2, CH), 0)
        process(k0 + 1, 1)
        return carry

    lax.fori_loop(0, CH // 2, body, 0)
    pltpu.make_async_copy(e_hbm.at[pl.ds(ebase, C)], ebuf0, se0).wait()

    plsc.subcore_barrier()
    pltpu.sync_copy(ssh.at[band], s_hbm.at[cid, band])


@functools.partial(
    pl.kernel,
    mesh=_mesh,
    compiler_params=pltpu.CompilerParams(needs_layout_passes=False),
    out_type=[jax.ShapeDtypeStruct((NC, NP, 128), F32)],
    scratch_types=[
        pltpu.VMEM((C,), F32),
        pltpu.VMEM((C,), F32),
        pltpu.VMEM((C,), F32),
        pltpu.VMEM((C,), F32),
        pltpu.VMEM((C,), F32),
        pltpu.VMEM((C,), F32),
        pltpu.VMEM((C,), F32),
        pltpu.VMEM((C,), F32),
        pltpu.VMEM((C, 128), F32),
        pltpu.VMEM((48,), F32),
        pltpu.VMEM((CH, C), I32),
        pltpu.VMEM_SHARED((NP, 128), F32),
        pltpu.SemaphoreType.DMA,
        pltpu.SemaphoreType.DMA,
    ],
)
def _sc2b(alpha_hbm, r2x_hbm, r2y_hbm, r2z_hbm, row_hbm, z128_hbm, mask_hbm,
          xacc_hbm, abuf0, pxb0, pyb0, pzb0, abuf1, pxb1, pyb1, pzb1,
          obuf, mv, idx2r, accsh, sq0, sq1):
    cid = lax.axis_index("c")
    sid = lax.axis_index("s")
    wid = sid * NC + cid

    band = pl.ds(sid * BAND, BAND)
    pltpu.sync_copy(z128_hbm.at[band], accsh.at[band])
    pltpu.sync_copy(z128_hbm.at[pl.ds(0, C)], obuf)
    pltpu.sync_copy(mask_hbm, mv)
    pltpu.sync_copy(row_hbm.at[pl.ds(wid * CH, CH)], idx2r)
    plsc.subcore_barrier()

    m1 = mv[0:16]
    m2 = mv[16:32]
    m3 = mv[32:48]

    ebase = wid * EPW
    bufs = ((abuf0, pxb0, pyb0, pzb0, sq0), (abuf1, pxb1, pyb1, pzb1, sq1))

    def start(k, b):
        ab, px, py, pz, sq = bufs[b]
        sl = pl.ds(ebase + k * C, C)
        pltpu.async_copy(alpha_hbm.at[sl], ab, sq)
        pltpu.async_copy(r2x_hbm.at[sl], px, sq)
        pltpu.async_copy(r2y_hbm.at[sl], py, sq)
        pltpu.async_copy(r2z_hbm.at[sl], pz, sq)

    def process(k, b):
        ab, px, py, pz, sq = bufs[b]
        sl = pl.ds(ebase + k * C, C)
        pltpu.make_async_copy(alpha_hbm.at[sl], ab, sq).wait()
        pltpu.make_async_copy(r2x_hbm.at[sl], px, sq).wait()
        pltpu.make_async_copy(r2y_hbm.at[sl], py, sq).wait()
        pltpu.make_async_copy(r2z_hbm.at[sl], pz, sq).wait()

        def edge(e, c2):
            ev = jnp.full((16,), e, I32)
            bx = plsc.load_gather(px, [ev])
            by = plsc.load_gather(py, [ev])
            bz = plsc.load_gather(pz, [ev])
            ba = plsc.load_gather(ab, [ev])
            obuf[e, 0:16] = ba * (bx * m1 + by * m2 + bz * m3)
            return c2

        lax.fori_loop(0, C, edge, 0)
        pltpu.sync_copy(obuf, accsh.at[idx2r.at[k]], add=True)

    start(0, 0)

    def body(i, carry):
        k0 = 2 * i
        start(k0 + 1, 1)
        process(k0, 0)
        start(jnp.remainder(k0 + 2, CH), 0)
        process(k0 + 1, 1)
        return carry

    lax.fori_loop(0, CH // 2, body, 0)
    sl0 = pl.ds(ebase, C)
    pltpu.make_async_copy(alpha_hbm.at[sl0], abuf0, sq0).wait()
    pltpu.make_async_copy(r2x_hbm.at[sl0], pxb0, sq0).wait()
    pltpu.make_async_copy(r2y_hbm.at[sl0], pyb0, sq0).wait()
    pltpu.make_async_copy(r2z_hbm.at[sl0], pzb0, sq0).wait()

    plsc.subcore_barrier()
    pltpu.sync_copy(accsh.at[band], xacc_hbm.at[cid, band])


BM = 1280
BE = 8192


def _bdot(a, b):
    return jnp.dot(a.astype(BF16), b.astype(BF16),
                   preferred_element_type=F32)


def _tck1_body(h_ref, we_ref, be_ref, ew1a_ref, ew1b_ref, eb1_ref,
               h1_ref, a_ref, b_ref):
    h1 = _bdot(h_ref[...], we_ref[...]) + be_ref[...]
    h1_ref[...] = h1
    heb1 = 0.5 * eb1_ref[...]
    a_ref[...] = _bdot(h1, ew1a_ref[...]) + heb1
    b_ref[...] = _bdot(h1, ew1b_ref[...]) + heb1


def _tck2_body(r_ref, ew2_ref, eb2_ref, cw1_ref, cb1_ref, cw2_ref, cb2_ref,
               e_ref, out_ref):
    e_ij = _bdot(r_ref[...], ew2_ref[...]) + eb2_ref[...]
    e_ref[...] = e_ij
    t = jnp.maximum(_bdot(e_ij, cw1_ref[...]) + cb1_ref[...], 0.0)
    a = _bdot(t, cw2_ref[...]) + cb2_ref[...]
    out_ref[...] = a.reshape(BE // 128, 128)


def _tck3_body(last, h_ref, xt_ref, s128_ref, xacc_ref,
               nw1a_ref, nw1b_ref, nb1_ref, nw2_ref,
               nb2_ref, ew1a_ref, ew1b_ref, eb1_ref, *out_refs):
    h = h_ref[...]
    m = s128_ref[0] + s128_ref[1]
    t = jnp.maximum(_bdot(h, nw1a_ref[...]) + _bdot(m, nw1b_ref[...])
                    + nb1_ref[...], 0.0)
    hn = h + _bdot(t, nw2_ref[...]) + nb2_ref[...]
    out_refs[0][...] = hn
    lane = lax.broadcasted_iota(I32, (1, 128), 1)
    mask = jnp.where((lane >= 1) & (lane <= 3), 1.0, 0.0).astype(F32)
    out_refs[1][...] = (xt_ref[...] + xacc_ref[0] + xacc_ref[1]) * mask
    if not last:
        heb1 = 0.5 * eb1_ref[...]
        out_refs[2][...] = _bdot(hn, ew1a_ref[...]) + heb1
        out_refs[3][...] = _bdot(hn, ew1b_ref[...]) + heb1


def _row_spec(bm, w):
    return pl.BlockSpec((bm, w), lambda i: (i, 0))


def _full_spec(shape):
    return pl.BlockSpec(shape, lambda i: tuple(0 for _ in shape))


def _pair_spec(bm, w):
    return pl.BlockSpec((NC, bm, w), lambda i: (0, i, 0))


def _tck1(h_pad, We, be, ew1a, ew1b, eb1):
    return pl.pallas_call(
        _tck1_body,
        grid=(NP // BM,),
        in_specs=[
            _row_spec(BM, 128),
            _full_spec((128, 128)), _full_spec((1, 128)),
            _full_spec((128, 128)), _full_spec((128, 128)),
            _full_spec((1, 128)),
        ],
        out_specs=[
            _row_spec(BM, 128), _row_spec(BM, 128), _row_spec(BM, 128),
        ],
        out_shape=[
            jax.ShapeDtypeStruct((NP, 128), F32),
            jax.ShapeDtypeStruct((NP, 128), F32),
            jax.ShapeDtypeStruct((NP, 128), F32),
        ],
    )(h_pad, We, be, ew1a, ew1b, eb1)


def _tck2(r1, ew2, eb2, cw1, cb1, cw2, cb2):
    return pl.pallas_call(
        _tck2_body,
        grid=(EP // BE,),
        in_specs=[
            _row_spec(BE, 128),
            _full_spec((128, 128)), _full_spec((1, 128)),
            _full_spec((128, 128)), _full_spec((1, 128)),
            _full_spec((128, 1)), _full_spec((1, 1)),
        ],
        out_specs=[_row_spec(BE, 128), _row_spec(BE // 128, 128)],
        out_shape=[
            jax.ShapeDtypeStruct((EP, 128), F32),
            jax.ShapeDtypeStruct((EP // 128, 128), F32),
        ],
    )(r1, ew2, eb2, cw1, cb1, cw2, cb2)


def _tck3(last, h1, xt, s128o, xacc, nw1a, nw1b, nb1, nw2,
          nb2, ew1a, ew1b, eb1):
    out_specs = [_row_spec(BM, 128), _row_spec(BM, 128)]
    out_shape = [
        jax.ShapeDtypeStruct((NP, 128), F32),
        jax.ShapeDtypeStruct((NP, 128), F32),
    ]
    if not last:
        out_specs += [_row_spec(BM, 128), _row_spec(BM, 128)]
        out_shape += [
            jax.ShapeDtypeStruct((NP, 128), F32),
            jax.ShapeDtypeStruct((NP, 128), F32),
        ]
    return pl.pallas_call(
        functools.partial(_tck3_body, last),
        grid=(NP // BM,),
        in_specs=[
            _row_spec(BM, 128), _row_spec(BM, 128),
            _pair_spec(BM, 128), _pair_spec(BM, 128),
            _full_spec((128, 128)), _full_spec((128, 128)),
            _full_spec((1, 128)), _full_spec((128, 128)),
            _full_spec((1, 128)),
            _full_spec((128, 128)), _full_spec((128, 128)),
            _full_spec((1, 128)),
        ],
        out_specs=out_specs,
        out_shape=out_shape,
    )(h1, xt, s128o, xacc, nw1a, nw1b, nb1, nw2, nb2,
      ew1a, ew1b, eb1)


def kernel(h, x, edge_index, We, be, ew1, eb1, ew2, eb2,
           nw1, nb1, nw2, nb2, cw1, cb1, cw2, cb2):
    row, col = edge_index[0], edge_index[1]
    rowp = jnp.concatenate([row, jnp.full((EP - E,), N, I32)]).reshape(NWK * CH, C)
    colp = jnp.concatenate([col, jnp.full((EP - E,), N, I32)]).reshape(NWK * CH, C)
    h_pad = jnp.pad(h, ((0, NP - N), (0, 0)))
    xt = jnp.pad(x, ((0, NP - N), (1, 124)))

    ew1a, ew1b, w1d = ew1[:H], ew1[H:2 * H], ew1[2 * H]
    nw1a, nw1b = nw1[:H], nw1[H:]
    be2 = be.reshape(1, 128)
    eb1_2 = eb1.reshape(1, 128)
    eb2_2 = eb2.reshape(1, 128)
    nb1_2 = nb1.reshape(1, 128)
    nb2_2 = nb2.reshape(1, 128)
    cb1_2 = cb1.reshape(1, 128)
    cb2_2 = cb2.reshape(1, 1)
    z128 = jnp.zeros((NP, 128), F32)
    lane_masks = jnp.zeros((48,), F32).at[jnp.array([1, 18, 35])].set(1.0)

    h1, A, B = _tck1(h_pad, We, be2, ew1a, ew1b, eb1_2)

    for layer in range(3):
        xt3 = xt[:, 1:4].reshape(NP * 3)
        r1, r2x, r2y, r2z = _sc1(A, B, xt3, rowp, colp, w1d)
        e_ij, alpha = _tck2(r1, ew2, eb2_2, cw1, cb1_2, cw2, cb2_2)
        s128o, = _sc2a(e_ij, rowp, z128)
        xacc, = _sc2b(alpha.reshape(EP), r2x, r2y, r2z, rowp, z128, lane_masks)
        last = layer == 2
        outs = _tck3(last, h1, xt, s128o, xacc,
                     nw1a, nw1b, nb1_2, nw2, nb2_2, ew1a, ew1b, eb1_2)
        if last:
            h1, xt = outs
        else:
            h1, xt, A, B = outs

    return h1[:N], xt[:N, 1:4]


# speedup vs baseline: 3.6998x; 1.0065x over previous
import functools

import jax
import jax.numpy as jnp
from jax import lax
from jax.experimental import pallas as pl
from jax.experimental.pallas import tpu as pltpu
from jax.experimental.pallas import tpu_sc as plsc

N = 10000
E = 320000
H = 128
NP = 10240
EP = 327680
NC = 2
NS = 16
NWK = NC * NS
EPW = EP // NWK
C = 128
CH = EPW // C
BAND = NP // NS
F32 = jnp.float32
BF16 = jnp.bfloat16
I32 = jnp.int32
U32 = jnp.uint32

_mesh = plsc.VectorSubcoreMesh(core_axis_name="c", subcore_axis_name="s")


def _bf16_round(v):
    u = plsc.bitcast(v, U32)
    lsb = (u >> 16) & jnp.uint32(1)
    u2 = (u + jnp.uint32(0x7FFF) + lsb) & jnp.uint32(0xFFFF0000)
    return plsc.bitcast(u2, F32)


@functools.partial(
    pl.kernel,
    mesh=_mesh,
    compiler_params=pltpu.CompilerParams(needs_layout_passes=False),
    out_type=[
        jax.ShapeDtypeStruct((EP, 128), F32),
        jax.ShapeDtypeStruct((NWK * CH, 4, C), F32),
    ],
    scratch_types=[
        pltpu.VMEM((C, 128), F32),
        pltpu.VMEM((C, 128), F32),
        pltpu.VMEM((C, 128), F32),
        pltpu.VMEM((C, 128), F32),
        pltpu.VMEM((NP * 3,), F32),
        pltpu.VMEM((4, C), F32),
        pltpu.VMEM((128,), F32),
        pltpu.VMEM((CH, C), I32),
        pltpu.VMEM((CH, C), I32),
        pltpu.SemaphoreType.DMA,
        pltpu.SemaphoreType.DMA,
        pltpu.SemaphoreType.DMA,
        pltpu.SemaphoreType.DMA,
    ],
)
def _sc1(a_hbm, b_hbm, xt3_hbm, row_hbm, col_hbm, w1d_hbm,
         r1_hbm, r2p_hbm,
         ga0, gb0, ga1, gb1, xt_tile, pbuf, w1dv,
         idx2r, idx2c, sa0, sb0, sa1, sb1):
    cid = lax.axis_index("c")
    sid = lax.axis_index("s")
    wid = sid * NC + cid

    pltpu.sync_copy(w1d_hbm, w1dv)
    pltpu.sync_copy(xt3_hbm, xt_tile)
    wblk = pl.ds(wid * CH, CH)
    pltpu.sync_copy(row_hbm.at[wblk], idx2r)
    pltpu.sync_copy(col_hbm.at[wblk], idx2c)
    for j in range(8):
        fs = pl.ds(j * 16, 16)
        w1dv[fs] = _bf16_round(w1dv[fs])

    ebase = wid * EPW
    bufs = ((ga0, gb0, sa0, sb0), (ga1, gb1, sa1, sb1))

    def start(k, b):
        ga, gb, sa, sb = bufs[b]
        pltpu.async_copy(a_hbm.at[idx2r.at[k]], ga, sa)
        pltpu.async_copy(b_hbm.at[idx2c.at[k]], gb, sb)

    def finish(k, b):
        ga, gb, sa, sb = bufs[b]
        pltpu.make_async_copy(a_hbm.at[idx2r.at[k]], ga, sa).wait()
        pltpu.make_async_copy(b_hbm.at[idx2c.at[k]], gb, sb).wait()

    def process(k, b):
        ga, gb, _, _ = bufs[b]
        for g in range(C // 16):
            gsl = pl.ds(g * 16, 16)
            rv = idx2r[k, gsl] * 3
            cv = idx2c[k, gsl] * 3
            px = plsc.load_gather(xt_tile, [rv]) - plsc.load_gather(xt_tile, [cv])
            py = plsc.load_gather(xt_tile, [rv + 1]) - plsc.load_gather(xt_tile, [cv + 1])
            pz = plsc.load_gather(xt_tile, [rv + 2]) - plsc.load_gather(xt_tile, [cv + 2])
            pbuf[0, gsl] = px
            pbuf[1, gsl] = py
            pbuf[2, gsl] = pz
            pbuf[3, gsl] = _bf16_round(px * px + py * py + pz * pz)

        finish(k, b)

        def edge(e, c2):
            dv = plsc.load_gather(pbuf.at[3], [jnp.full((16,), e, I32)])
            for j in range(8):
                fs = pl.ds(j * 16, 16)
                z = ga[e, fs] + gb[e, fs] + dv * w1dv[fs]
                ga[e, fs] = jnp.maximum(z, 0.0)
            return c2

        lax.fori_loop(0, C, edge, 0)

        pltpu.sync_copy(ga, r1_hbm.at[pl.ds(ebase + k * C, C)])
        pltpu.sync_copy(pbuf, r2p_hbm.at[wid * CH + k])

    start(0, 0)

    def body(i, carry):
        k0 = 2 * i
        start(k0 + 1, 1)
        process(k0, 0)
        start(jnp.remainder(k0 + 2, CH), 0)
        process(k0 + 1, 1)
        return carry

    lax.fori_loop(0, CH // 2, body, 0)
    finish(0, 0)


@functools.partial(
    pl.kernel,
    mesh=_mesh,
    compiler_params=pltpu.CompilerParams(needs_layout_passes=False),
    out_type=[jax.ShapeDtypeStruct((NC, NP, 128), F32)],
    scratch_types=[
        pltpu.VMEM((C, 128), F32),
        pltpu.VMEM((C, 128), F32),
        pltpu.VMEM((CH, C), I32),
        pltpu.VMEM_SHARED((NP, 128), F32),
        pltpu.SemaphoreType.DMA,
        pltpu.SemaphoreType.DMA,
    ],
)
def _sc2a(e_hbm, row_hbm, z128_hbm, s_hbm, ebuf0, ebuf1, idx2r, ssh, se0, se1):
    cid = lax.axis_index("c")
    sid = lax.axis_index("s")
    wid = sid * NC + cid

    band = pl.ds(sid * BAND, BAND)
    pltpu.sync_copy(z128_hbm.at[band], ssh.at[band])
    pltpu.sync_copy(row_hbm.at[pl.ds(wid * CH, CH)], idx2r)
    plsc.subcore_barrier()

    ebase = wid * EPW
    bufs = ((ebuf0, se0), (ebuf1, se1))

    def start(k, b):
        eb, se = bufs[b]
        pltpu.async_copy(e_hbm.at[pl.ds(ebase + k * C, C)], eb, se)

    def process(k, b):
        eb, se = bufs[b]
        pltpu.make_async_copy(e_hbm.at[pl.ds(ebase + k * C, C)], eb, se).wait()
        pltpu.sync_copy(eb, ssh.at[idx2r.at[k]], add=True)

    start(0, 0)

    def body(i, carry):
        k0 = 2 * i
        start(k0 + 1, 1)
        process(k0, 0)
        start(jnp.remainder(k0 + 2, CH), 0)
        process(k0 + 1, 1)
        return carry

    lax.fori_loop(0, CH // 2, body, 0)
    pltpu.make_async_copy(e_hbm.at[pl.ds(ebase, C)], ebuf0, se0).wait()

    plsc.subcore_barrier()
    pltpu.sync_copy(ssh.at[band], s_hbm.at[cid, band])


@functools.partial(
    pl.kernel,
    mesh=_mesh,
    compiler_params=pltpu.CompilerParams(needs_layout_passes=False),
    out_type=[jax.ShapeDtypeStruct((NC, NP, 128), F32)],
    scratch_types=[
        pltpu.VMEM((C,), F32),
        pltpu.VMEM((4, C), F32),
        pltpu.VMEM((C,), F32),
        pltpu.VMEM((4, C), F32),
        pltpu.VMEM((C, 128), F32),
        pltpu.VMEM((48,), F32),
        pltpu.VMEM((CH, C), I32),
        pltpu.VMEM_SHARED((NP, 128), F32),
        pltpu.SemaphoreType.DMA,
        pltpu.SemaphoreType.DMA,
    ],
)
def _sc2b(alpha_hbm, r2p_hbm, row_hbm, z128_hbm, mask_hbm,
          xacc_hbm, abuf0, pb0, abuf1, pb1,
          obuf, mv, idx2r, accsh, sq0, sq1):
    cid = lax.axis_index("c")
    sid = lax.axis_index("s")
    wid = sid * NC + cid

    band = pl.ds(sid * BAND, BAND)
    pltpu.sync_copy(z128_hbm.at[band], accsh.at[band])
    pltpu.sync_copy(z128_hbm.at[pl.ds(0, C)], obuf)
    pltpu.sync_copy(mask_hbm, mv)
    pltpu.sync_copy(row_hbm.at[pl.ds(wid * CH, CH)], idx2r)
    plsc.subcore_barrier()

    m1 = mv[0:16]
    m2 = mv[16:32]
    m3 = mv[32:48]

    ebase = wid * EPW
    bufs = ((abuf0, pb0, sq0), (abuf1, pb1, sq1))

    def start(k, b):
        ab, pb, sq = bufs[b]
        pltpu.async_copy(alpha_hbm.at[pl.ds(ebase + k * C, C)], ab, sq)
        pltpu.async_copy(r2p_hbm.at[wid * CH + k], pb, sq)

    def process(k, b):
        ab, pb, sq = bufs[b]
        pltpu.make_async_copy(alpha_hbm.at[pl.ds(ebase + k * C, C)], ab, sq).wait()
        pltpu.make_async_copy(r2p_hbm.at[wid * CH + k], pb, sq).wait()

        def edge(e, c2):
            ev = jnp.full((16,), e, I32)
            bx = plsc.load_gather(pb.at[0], [ev])
            by = plsc.load_gather(pb.at[1], [ev])
            bz = plsc.load_gather(pb.at[2], [ev])
            ba = plsc.load_gather(ab, [ev])
            obuf[e, 0:16] = ba * (bx * m1 + by * m2 + bz * m3)
            return c2

        lax.fori_loop(0, C, edge, 0)
        pltpu.sync_copy(obuf, accsh.at[idx2r.at[k]], add=True)

    start(0, 0)

    def body(i, carry):
        k0 = 2 * i
        start(k0 + 1, 1)
        process(k0, 0)
        start(jnp.remainder(k0 + 2, CH), 0)
        process(k0 + 1, 1)
        return carry

    lax.fori_loop(0, CH // 2, body, 0)
    pltpu.make_async_copy(alpha_hbm.at[pl.ds(ebase, C)], abuf0, sq0).wait()
    pltpu.make_async_copy(r2p_hbm.at[wid * CH], pb0, sq0).wait()

    plsc.subcore_barrier()
    pltpu.sync_copy(accsh.at[band], xacc_hbm.at[cid, band])


BM = 1280
BE = 8192


def _bdot(a, b):
    return jnp.dot(a.astype(BF16), b.astype(BF16),
                   preferred_element_type=F32)


def _tck1_body(h_ref, we_ref, be_ref, ew1a_ref, ew1b_ref, eb1_ref,
               h1_ref, a_ref, b_ref):
    h1 = _bdot(h_ref[...], we_ref[...]) + be_ref[...]
    h1_ref[...] = h1
    heb1 = 0.5 * eb1_ref[...]
    a_ref[...] = _bdot(h1, ew1a_ref[...]) + heb1
    b_ref[...] = _bdot(h1, ew1b_ref[...]) + heb1


def _tck2_body(r_ref, ew2_ref, eb2_ref, cw1_ref, cb1_ref, cw2_ref, cb2_ref,
               e_ref, out_ref):
    e_ij = _bdot(r_ref[...], ew2_ref[...]) + eb2_ref[...]
    e_ref[...] = e_ij
    t = jnp.maximum(_bdot(e_ij, cw1_ref[...]) + cb1_ref[...], 0.0)
    a = _bdot(t, cw2_ref[...]) + cb2_ref[...]
    out_ref[...] = a.reshape(BE // 128, 128)


def _tck3_body(last, h_ref, xt_ref, s128_ref, xacc_ref,
               nw1a_ref, nw1b_ref, nb1_ref, nw2_ref,
               nb2_ref, ew1a_ref, ew1b_ref, eb1_ref, *out_refs):
    h = h_ref[...]
    m = s128_ref[0] + s128_ref[1]
    t = jnp.maximum(_bdot(h, nw1a_ref[...]) + _bdot(m, nw1b_ref[...])
                    + nb1_ref[...], 0.0)
    hn = h + _bdot(t, nw2_ref[...]) + nb2_ref[...]
    out_refs[0][...] = hn
    lane = lax.broadcasted_iota(I32, (1, 128), 1)
    mask = jnp.where((lane >= 1) & (lane <= 3), 1.0, 0.0).astype(F32)
    out_refs[1][...] = (xt_ref[...] + xacc_ref[0] + xacc_ref[1]) * mask
    if not last:
        heb1 = 0.5 * eb1_ref[...]
        out_refs[2][...] = _bdot(hn, ew1a_ref[...]) + heb1
        out_refs[3][...] = _bdot(hn, ew1b_ref[...]) + heb1


def _row_spec(bm, w):
    return pl.BlockSpec((bm, w), lambda i: (i, 0))


def _full_spec(shape):
    return pl.BlockSpec(shape, lambda i: tuple(0 for _ in shape))


def _pair_spec(bm, w):
    return pl.BlockSpec((NC, bm, w), lambda i: (0, i, 0))


def _tck1(h_pad, We, be, ew1a, ew1b, eb1):
    return pl.pallas_call(
        _tck1_body,
        grid=(NP // BM,),
        in_specs=[
            _row_spec(BM, 128),
            _full_spec((128, 128)), _full_spec((1, 128)),
            _full_spec((128, 128)), _full_spec((128, 128)),
            _full_spec((1, 128)),
        ],
        out_specs=[
            _row_spec(BM, 128), _row_spec(BM, 128), _row_spec(BM, 128),
        ],
        out_shape=[
            jax.ShapeDtypeStruct((NP, 128), F32),
            jax.ShapeDtypeStruct((NP, 128), F32),
            jax.ShapeDtypeStruct((NP, 128), F32),
        ],
    )(h_pad, We, be, ew1a, ew1b, eb1)


def _tck2(r1, ew2, eb2, cw1, cb1, cw2, cb2):
    return pl.pallas_call(
        _tck2_body,
        grid=(EP // BE,),
        in_specs=[
            _row_spec(BE, 128),
            _full_spec((128, 128)), _full_spec((1, 128)),
            _full_spec((128, 128)), _full_spec((1, 128)),
            _full_spec((128, 1)), _full_spec((1, 1)),
        ],
        out_specs=[_row_spec(BE, 128), _row_spec(BE // 128, 128)],
        out_shape=[
            jax.ShapeDtypeStruct((EP, 128), F32),
            jax.ShapeDtypeStruct((EP // 128, 128), F32),
        ],
    )(r1, ew2, eb2, cw1, cb1, cw2, cb2)


def _tck3(last, h1, xt, s128o, xacc, nw1a, nw1b, nb1, nw2,
          nb2, ew1a, ew1b, eb1):
    out_specs = [_row_spec(BM, 128), _row_spec(BM, 128)]
    out_shape = [
        jax.ShapeDtypeStruct((NP, 128), F32),
        jax.ShapeDtypeStruct((NP, 128), F32),
    ]
    if not last:
        out_specs += [_row_spec(BM, 128), _row_spec(BM, 128)]
        out_shape += [
            jax.ShapeDtypeStruct((NP, 128), F32),
            jax.ShapeDtypeStruct((NP, 128), F32),
        ]
    return pl.pallas_call(
        functools.partial(_tck3_body, last),
        grid=(NP // BM,),
        in_specs=[
            _row_spec(BM, 128), _row_spec(BM, 128),
            _pair_spec(BM, 128), _pair_spec(BM, 128),
            _full_spec((128, 128)), _full_spec((128, 128)),
            _full_spec((1, 128)), _full_spec((128, 128)),
            _full_spec((1, 128)),
            _full_spec((128, 128)), _full_spec((128, 128)),
            _full_spec((1, 128)),
        ],
        out_specs=out_specs,
        out_shape=out_shape,
    )(h1, xt, s128o, xacc, nw1a, nw1b, nb1, nw2, nb2,
      ew1a, ew1b, eb1)


def kernel(h, x, edge_index, We, be, ew1, eb1, ew2, eb2,
           nw1, nb1, nw2, nb2, cw1, cb1, cw2, cb2):
    row, col = edge_index[0], edge_index[1]
    rowp = jnp.concatenate([row, jnp.full((EP - E,), N, I32)]).reshape(NWK * CH, C)
    colp = jnp.concatenate([col, jnp.full((EP - E,), N, I32)]).reshape(NWK * CH, C)
    h_pad = jnp.pad(h, ((0, NP - N), (0, 0)))
    xt = jnp.pad(x, ((0, NP - N), (1, 124)))

    ew1a, ew1b, w1d = ew1[:H], ew1[H:2 * H], ew1[2 * H]
    nw1a, nw1b = nw1[:H], nw1[H:]
    be2 = be.reshape(1, 128)
    eb1_2 = eb1.reshape(1, 128)
    eb2_2 = eb2.reshape(1, 128)
    nb1_2 = nb1.reshape(1, 128)
    nb2_2 = nb2.reshape(1, 128)
    cb1_2 = cb1.reshape(1, 128)
    cb2_2 = cb2.reshape(1, 1)
    z128 = jnp.zeros((NP, 128), F32)
    lane_masks = jnp.zeros((48,), F32).at[jnp.array([1, 18, 35])].set(1.0)

    h1, A, B = _tck1(h_pad, We, be2, ew1a, ew1b, eb1_2)

    for layer in range(3):
        xt3 = xt[:, 1:4].reshape(NP * 3)
        r1, r2p = _sc1(A, B, xt3, rowp, colp, w1d)
        e_ij, alpha = _tck2(r1, ew2, eb2_2, cw1, cb1_2, cw2, cb2_2)
        s128o, = _sc2a(e_ij, rowp, z128)
        xacc, = _sc2b(alpha.reshape(EP), r2p, rowp, z128, lane_masks)
        last = layer == 2
        outs = _tck3(last, h1, xt, s128o, xacc,
                     nw1a, nw1b, nb1_2, nw2, nb2_2, ew1a, ew1b, eb1_2)
        if last:
            h1, xt = outs
        else:
            h1, xt, A, B = outs

    return h1[:N], xt[:N, 1:4]
